# baseline (speedup 1.0000x reference)
import functools

import jax
import jax.numpy as jnp
from jax import lax
from jax.experimental import pallas as pl
from jax.experimental.pallas import tpu as pltpu
from jax.experimental.pallas import tpu_sc as plsc

_OVERLAP = 0.45
_TOP_K = 200
_N = 20000
_ROWS = 160
_LANES = 128
_NPAD = _ROWS * _LANES
_BROWS = 640
_IMIN = jnp.iinfo(jnp.int32).min
_NW = 16
_SH = _NPAD // _NW
_SHV = _SH // 16
_PACK = 2048
_OSB, _OID, _OX1, _OY1, _OX2, _OY2, _OAR = 16, 272, 528, 784, 1040, 1296, 1552
_KPAD = 208
_KV = _KPAD // 16


def _tc_kernel(locI_ref, sco_ref, dbxI_ref, boxI_ref, sbit_ref, tj_ref):
    f32 = jnp.float32
    ll = locI_ref[...]
    dd = dbxI_ref[...]
    lane = lax.broadcasted_iota(jnp.int32, (_BROWS, _LANES), 1)
    m01 = jax.lax.bitwise_and(lane, 3) < 2
    dwh = pltpu.roll(dd, -2 % _LANES, 1)
    a = jnp.where(m01, dd + (ll * f32(0.1)) * dwh,
                  dd * jnp.exp(ll * f32(0.2)))
    wh01 = pltpu.roll(a, -2 % _LANES, 1)
    b = a - wh01 / f32(2.0)
    x12 = pltpu.roll(b, 2, 1)
    boxI_ref[...] = jnp.where(m01, b, x12 + a)

    bits = pltpu.bitcast(sco_ref[...], jnp.int32)
    sbit_ref[...] = bits
    flat = lax.broadcasted_iota(jnp.int32, (_ROWS, _LANES), 0) * _LANES + \
        lax.broadcasted_iota(jnp.int32, (_ROWS, _LANES), 1)

    def bs1(_, c):
        lo, hi = c
        mid = lo + ((hi - lo + jnp.int32(1)) >> 1)
        cnt = jnp.sum((bits >= mid).astype(jnp.int32))
        ok = cnt >= _TOP_K
        return jnp.where(ok, mid, lo), jnp.where(ok, hi, mid - 1)

    tb, _ = lax.fori_loop(0, 31, bs1, (jnp.int32(0), jnp.int32(0x3F7FFFFF)))
    k2 = _TOP_K - jnp.sum((bits > tb).astype(jnp.int32))

    def bs2(_, c):
        lo, hi = c
        mid = lo + ((hi - lo + jnp.int32(1)) >> 1)
        cnt = jnp.sum(((bits == tb) & (flat >= mid)).astype(jnp.int32))
        ok = cnt >= k2
        return jnp.where(ok, mid, lo), jnp.where(ok, hi, mid - 1)

    jj, _ = lax.fori_loop(0, 16, bs2, (jnp.int32(0), jnp.int32(_NPAD - 1)))
    tj_ref[0, 0] = tb
    tj_ref[0, 1] = jj


def _sc_kernel(sb_h, bI_h, tj_h, keep_h, cnt_h,
               sb_v, bI_v, tj_v, pack_v, cntv, keep_v, spbig, big_v,
               ksb, kid, kx1, ky1, kx2, ky2, kar):
    f32 = jnp.float32
    i32 = jnp.int32
    wid = lax.axis_index("s")
    base = wid * _SH

    pltpu.sync_copy(sb_h.at[pl.ds(base, _SH)], sb_v)
    pltpu.sync_copy(bI_h.at[pl.ds(base * 4, _SH * 4)], bI_v)
    pltpu.sync_copy(tj_h, tj_v)
    tjv = tj_v[pl.ds(0, 16)]
    tb = tjv[0]
    jj = tjv[1]

    z16 = jnp.zeros((16,), i32)
    for v in range(_SHV + 1):
        keep_v[pl.ds(v * 16, 16)] = z16

    @pl.when(wid != 0)
    def _zero_keep():
        pltpu.sync_copy(keep_v.at[pl.ds(0, _SH)],
                        keep_h.at[pl.ds(base, _SH)])

    cnt = i32(0)
    for v in range(_SHV):
        sl = pl.ds(v * 16, 16)
        bits = sb_v[sl]
        gi = lax.iota(i32, 16) + (base + v * 16)
        selm = (bits > tb) | ((bits == tb) & (gi >= jj))
        pref = plsc.cumsum(selm.astype(i32))
        tgt = jnp.where(selm, cnt + pref - 1, 255)
        li4 = (lax.iota(i32, 16) + v * 16) * 4
        x1 = plsc.load_gather(bI_v, [li4])
        y1 = plsc.load_gather(bI_v, [li4 + 1])
        x2 = plsc.load_gather(bI_v, [li4 + 2])
        y2 = plsc.load_gather(bI_v, [li4 + 3])
        plsc.store_scatter(pack_v, [tgt + _OSB], bits)
        plsc.store_scatter(pack_v, [tgt + _OID], gi)
        plsc.store_scatter(pack_v, [tgt + _OX1], plsc.bitcast(x1, i32))
        plsc.store_scatter(pack_v, [tgt + _OY1], plsc.bitcast(y1, i32))
        plsc.store_scatter(pack_v, [tgt + _OX2], plsc.bitcast(x2, i32))
        plsc.store_scatter(pack_v, [tgt + _OY2], plsc.bitcast(y2, i32))
        plsc.store_scatter(pack_v, [tgt + _OAR],
                           plsc.bitcast((x2 - x1) * (y2 - y1), i32))
        cnt = cnt + pref[15]

    lane = lax.iota(i32, 16)
    pack_v[pl.ds(0, 16)] = jnp.where(lane == 0, cnt, 0)
    pltpu.sync_copy(pack_v, spbig.at[wid])
    plsc.subcore_barrier()

    @pl.when(wid == 0)
    def _phase_b():
        pltpu.sync_copy(spbig, big_v)
        cvec = plsc.load_gather(
            big_v, [lax.iota(i32, 16), jnp.zeros((16,), i32)])
        offs = []
        run = i32(0)
        for w in range(_NW):
            offs.append(run)
            run = run + cvec[w]

        for k in range(_KV):
            qv = lax.iota(i32, 16) + (k * 16)
            wq = jnp.zeros((16,), i32)
            bq = jnp.zeros((16,), i32)
            for w in range(1, _NW):
                m = qv >= offs[w]
                wq = wq + m.astype(i32)
                bq = jnp.where(m, offs[w], bq)
            lq = jnp.minimum(jnp.maximum(qv - bq, 0), 255)
            valid = qv < _TOP_K
            s_k = plsc.load_gather(big_v, [wq, lq + _OSB])
            i_k = plsc.load_gather(big_v, [wq, lq + _OID])
            sl = pl.ds(k * 16, 16)
            ksb[sl] = jnp.where(valid, s_k, _IMIN)
            kid[sl] = jnp.where(valid, i_k, 0)
            kx1[sl] = plsc.bitcast(
                plsc.load_gather(big_v, [wq, lq + _OX1]), f32)
            ky1[sl] = plsc.bitcast(
                plsc.load_gather(big_v, [wq, lq + _OY1]), f32)
            kx2[sl] = plsc.bitcast(
                plsc.load_gather(big_v, [wq, lq + _OX2]), f32)
            ky2[sl] = plsc.bitcast(
                plsc.load_gather(big_v, [wq, lq + _OY2]), f32)
            kar[sl] = plsc.bitcast(
                plsc.load_gather(big_v, [wq, lq + _OAR]), f32)

        def cond(st):
            return st[1]

        def body(st):
            count, _, mm = st
            accp = jnp.full((16,), -1, i32)
            for k in range(_KV):
                sl = pl.ds(k * 16, 16)
                qv = lax.iota(i32, 16) + (k * 16)
                accp = jnp.maximum(accp, jnp.where(ksb[sl] == mm, qv, -1))
            p = jnp.max(accp)
            pv = jnp.zeros((16,), i32) + p
            ii = plsc.load_gather(kid, [pv])[0]
            bx1 = plsc.load_gather(kx1, [pv])[0]
            by1 = plsc.load_gather(ky1, [pv])[0]
            bx2 = plsc.load_gather(kx2, [pv])[0]
            by2 = plsc.load_gather(ky2, [pv])[0]
            bar = plsc.load_gather(kar, [pv])[0]
            acc = jnp.full((16,), _IMIN, i32)
            for k in range(_KV):
                sl = pl.ds(k * 16, 16)
                tw = jnp.maximum(
                    jnp.minimum(kx2[sl], bx2) - jnp.maximum(kx1[sl], bx1),
                    f32(0.0))
                th = jnp.maximum(
                    jnp.minimum(ky2[sl], by2) - jnp.maximum(ky1[sl], by1),
                    f32(0.0))
                inter = tw * th
                iou = inter / (kar[sl] - inter + bar)
                nk = jnp.where(iou <= f32(_OVERLAP), ksb[sl], _IMIN)
                ksb[sl] = nk
                acc = jnp.maximum(acc, nk)
            mm2 = jnp.max(acc)
            cv = jnp.where(lax.iota(i32, 16) == 0, count, _SH)
            iv = jnp.zeros((16,), i32) + ii
            plsc.store_scatter(keep_v, [cv], iv)
            return count + 1, mm2 > _IMIN, mm2

        acc0 = jnp.full((16,), _IMIN, i32)
        for k in range(_KV):
            acc0 = jnp.maximum(acc0, ksb[pl.ds(k * 16, 16)])
        mm0 = jnp.max(acc0)
        count, _, _ = lax.while_loop(cond, body, (i32(0), True, mm0))
        cntv[pl.ds(0, 16)] = jnp.where(lax.iota(i32, 16) == 0, count, 0)
        pltpu.sync_copy(cntv, cnt_h)
        pltpu.sync_copy(keep_v.at[pl.ds(0, _SH)],
                        keep_h.at[pl.ds(0, _SH)])


_SC_SCRATCH = [
        pltpu.VMEM((_SH,), jnp.int32),
        pltpu.VMEM((_SH * 4,), jnp.float32),
        pltpu.VMEM((16,), jnp.int32),
        pltpu.VMEM((_PACK,), jnp.int32),
        pltpu.VMEM((16,), jnp.int32),
        pltpu.VMEM((_SH + 16,), jnp.int32),
        pltpu.VMEM_SHARED((_NW, _PACK), jnp.int32),
        pltpu.VMEM((_NW, _PACK), jnp.int32),
        pltpu.VMEM((_KPAD,), jnp.int32),
        pltpu.VMEM((_KPAD,), jnp.int32),
        pltpu.VMEM((_KPAD,), jnp.float32),
        pltpu.VMEM((_KPAD,), jnp.float32),
        pltpu.VMEM((_KPAD,), jnp.float32),
        pltpu.VMEM((_KPAD,), jnp.float32),
        pltpu.VMEM((_KPAD,), jnp.float32),
]


@functools.lru_cache(maxsize=1)
def _make_sc_call():
  return functools.partial(
    pl.kernel,
    out_type=(
        jax.ShapeDtypeStruct((_NPAD,), jnp.int32),
        jax.ShapeDtypeStruct((16,), jnp.int32),
    ),
    mesh=plsc.VectorSubcoreMesh(core_axis_name="c", subcore_axis_name="s",
                                num_cores=1),
    compiler_params=pltpu.CompilerParams(needs_layout_passes=False),
    scratch_types=_SC_SCRATCH,
  )(_sc_kernel)


def kernel(loc, scores, dbox_list):
    f32 = jnp.float32
    locI = jnp.zeros((_NPAD, 4), f32).at[:_N].set(loc).reshape(
        _BROWS, _LANES)
    dbxI = jnp.zeros((_NPAD, 4), f32).at[:_N].set(dbox_list).reshape(
        _BROWS, _LANES)
    scop = jnp.full((_NPAD,), -jnp.inf, f32).at[:_N].set(scores).reshape(
        _ROWS, _LANES)
    boxI, sbit, tj = pl.pallas_call(
        _tc_kernel,
        out_shape=(
            jax.ShapeDtypeStruct((_BROWS, _LANES), f32),
            jax.ShapeDtypeStruct((_ROWS, _LANES), jnp.int32),
            jax.ShapeDtypeStruct((1, 2), jnp.int32),
        ),
        out_specs=(
            pl.BlockSpec(),
            pl.BlockSpec(),
            pl.BlockSpec(memory_space=pltpu.SMEM),
        ),
    )(locI, scop, dbxI)
    bI = boxI.reshape(_NPAD, 4)
    tj16 = jnp.zeros((16,), jnp.int32).at[0].set(tj[0, 0]).at[1].set(tj[0, 1])
    keep_p, cnt16 = _make_sc_call()(
        sbit.reshape(_NPAD), boxI.reshape(_NPAD * 4), tj16)
    return bI[:_N], keep_p[:_N], cnt16[0]


# speedup vs baseline: 1.7241x; 1.7241x over previous
import functools

import jax
import jax.numpy as jnp
from jax import lax
from jax.experimental import pallas as pl
from jax.experimental.pallas import tpu as pltpu
from jax.experimental.pallas import tpu_sc as plsc

_OVERLAP = 0.45
_TOP_K = 200
_N = 20000
_ROWS = 160
_LANES = 128
_NPAD = _ROWS * _LANES
_IMIN = jnp.iinfo(jnp.int32).min
_NW = 16
_SH = _NPAD // _NW
_SHV = _SH // 16
_PACK = 2048
_OSB, _OID, _OX1, _OY1, _OX2, _OY2, _OAR = 16, 272, 528, 784, 1040, 1296, 1552
_KPAD = 208
_KV = _KPAD // 16


def _tc_kernel(loc_ref, sco_ref, dbox_ref, box_ref, sbit_ref, tj_ref):
    f32 = jnp.float32
    l0 = loc_ref[0]
    l1 = loc_ref[1]
    l2 = loc_ref[2]
    l3 = loc_ref[3]
    d0 = dbox_ref[0]
    d1 = dbox_ref[1]
    d2 = dbox_ref[2]
    d3 = dbox_ref[3]
    cx = d0 + (l0 * f32(0.1)) * d2
    cy = d1 + (l1 * f32(0.1)) * d3
    w = d2 * jnp.exp(l2 * f32(0.2))
    h = d3 * jnp.exp(l3 * f32(0.2))
    x1 = cx - w / f32(2.0)
    y1 = cy - h / f32(2.0)
    box_ref[0] = x1
    box_ref[1] = y1
    box_ref[2] = x1 + w
    box_ref[3] = y1 + h

    bits = pltpu.bitcast(sco_ref[...], jnp.int32)
    sbit_ref[...] = bits
    flat = lax.broadcasted_iota(jnp.int32, (_ROWS, _LANES), 0) * _LANES + \
        lax.broadcasted_iota(jnp.int32, (_ROWS, _LANES), 1)

    def bs1(_, c):
        lo, hi = c
        mid = lo + ((hi - lo + jnp.int32(1)) >> 1)
        cnt = jnp.sum((bits >= mid).astype(jnp.int32))
        ok = cnt >= _TOP_K
        return jnp.where(ok, mid, lo), jnp.where(ok, hi, mid - 1)

    tb, _ = lax.fori_loop(0, 31, bs1, (jnp.int32(0), jnp.int32(0x3F7FFFFF)))
    k2 = _TOP_K - jnp.sum((bits > tb).astype(jnp.int32))

    def bs2(_, c):
        lo, hi = c
        mid = lo + ((hi - lo + jnp.int32(1)) >> 1)
        cnt = jnp.sum(((bits == tb) & (flat >= mid)).astype(jnp.int32))
        ok = cnt >= k2
        return jnp.where(ok, mid, lo), jnp.where(ok, hi, mid - 1)

    jj, _ = lax.fori_loop(0, 16, bs2, (jnp.int32(0), jnp.int32(_NPAD - 1)))
    tj_ref[0, 0] = tb
    tj_ref[0, 1] = jj


def _sc_kernel(sb_h, x1_h, y1_h, x2_h, y2_h, tj_h, keep_h, cnt_h,
               sb_v, x1_v, y1_v, x2_v, y2_v, tj_v,
               pack_v, cntv, keep_v, spbig, big_v,
               ksb, kid, kx1, ky1, kx2, ky2, kar):
    f32 = jnp.float32
    i32 = jnp.int32
    wid = lax.axis_index("s")
    base = wid * _SH

    pltpu.sync_copy(sb_h.at[pl.ds(base, _SH)], sb_v)
    pltpu.sync_copy(x1_h.at[pl.ds(base, _SH)], x1_v)
    pltpu.sync_copy(y1_h.at[pl.ds(base, _SH)], y1_v)
    pltpu.sync_copy(x2_h.at[pl.ds(base, _SH)], x2_v)
    pltpu.sync_copy(y2_h.at[pl.ds(base, _SH)], y2_v)
    pltpu.sync_copy(tj_h, tj_v)
    tjv = tj_v[pl.ds(0, 16)]
    tb = tjv[0]
    jj = tjv[1]

    z16 = jnp.zeros((16,), i32)
    for v in range(_SHV + 1):
        keep_v[pl.ds(v * 16, 16)] = z16

    @pl.when(wid != 0)
    def _zero_keep():
        pltpu.sync_copy(keep_v.at[pl.ds(0, _SH)],
                        keep_h.at[pl.ds(base, _SH)])

    cnt = i32(0)
    for v in range(_SHV):
        sl = pl.ds(v * 16, 16)
        bits = sb_v[sl]
        gi = lax.iota(i32, 16) + (base + v * 16)
        selm = (bits > tb) | ((bits == tb) & (gi >= jj))
        x1 = x1_v[sl]
        y1 = y1_v[sl]
        x2 = x2_v[sl]
        y2 = y2_v[sl]
        pref = plsc.cumsum(selm.astype(i32))
        tgt = jnp.where(selm, cnt + pref - 1, 255)
        plsc.store_scatter(pack_v, [tgt + _OSB], bits)
        plsc.store_scatter(pack_v, [tgt + _OID], gi)
        plsc.store_scatter(pack_v, [tgt + _OX1], plsc.bitcast(x1, i32))
        plsc.store_scatter(pack_v, [tgt + _OY1], plsc.bitcast(y1, i32))
        plsc.store_scatter(pack_v, [tgt + _OX2], plsc.bitcast(x2, i32))
        plsc.store_scatter(pack_v, [tgt + _OY2], plsc.bitcast(y2, i32))
        plsc.store_scatter(pack_v, [tgt + _OAR],
                           plsc.bitcast((x2 - x1) * (y2 - y1), i32))
        cnt = cnt + pref[15]

    lane = lax.iota(i32, 16)
    pack_v[pl.ds(0, 16)] = jnp.where(lane == 0, cnt, 0)
    pltpu.sync_copy(pack_v, spbig.at[wid])
    plsc.subcore_barrier()

    @pl.when(wid == 0)
    def _phase_b():
        pltpu.sync_copy(spbig, big_v)
        cvec = plsc.load_gather(
            big_v, [lax.iota(i32, 16), jnp.zeros((16,), i32)])
        offs = []
        run = i32(0)
        for w in range(_NW):
            offs.append(run)
            run = run + cvec[w]

        for k in range(_KV):
            qv = lax.iota(i32, 16) + (k * 16)
            wq = jnp.zeros((16,), i32)
            bq = jnp.zeros((16,), i32)
            for w in range(1, _NW):
                m = qv >= offs[w]
                wq = wq + m.astype(i32)
                bq = jnp.where(m, offs[w], bq)
            lq = jnp.minimum(jnp.maximum(qv - bq, 0), 255)
            valid = qv < _TOP_K
            s_k = plsc.load_gather(big_v, [wq, lq + _OSB])
            i_k = plsc.load_gather(big_v, [wq, lq + _OID])
            sl = pl.ds(k * 16, 16)
            ksb[sl] = jnp.where(valid, s_k, _IMIN)
            kid[sl] = jnp.where(valid, i_k, -1)
            kx1[sl] = plsc.bitcast(plsc.load_gather(big_v, [wq, lq + _OX1]), f32)
            ky1[sl] = plsc.bitcast(plsc.load_gather(big_v, [wq, lq + _OY1]), f32)
            kx2[sl] = plsc.bitcast(plsc.load_gather(big_v, [wq, lq + _OX2]), f32)
            ky2[sl] = plsc.bitcast(plsc.load_gather(big_v, [wq, lq + _OY2]), f32)
            kar[sl] = plsc.bitcast(plsc.load_gather(big_v, [wq, lq + _OAR]), f32)

        def cond(st):
            return st[1]

        def body(st):
            count, _, mm = st
            accp = jnp.full((16,), -1, i32)
            for k in range(_KV):
                sl = pl.ds(k * 16, 16)
                qv = lax.iota(i32, 16) + (k * 16)
                accp = jnp.maximum(accp, jnp.where(ksb[sl] == mm, qv, -1))
            p = jnp.max(accp)
            pv = jnp.zeros((16,), i32) + p
            ii = plsc.load_gather(kid, [pv])[0]
            bx1 = plsc.load_gather(kx1, [pv])[0]
            by1 = plsc.load_gather(ky1, [pv])[0]
            bx2 = plsc.load_gather(kx2, [pv])[0]
            by2 = plsc.load_gather(ky2, [pv])[0]
            bar = plsc.load_gather(kar, [pv])[0]
            acc = jnp.full((16,), _IMIN, i32)
            for k in range(_KV):
                sl = pl.ds(k * 16, 16)
                tw = jnp.maximum(
                    jnp.minimum(kx2[sl], bx2) - jnp.maximum(kx1[sl], bx1),
                    f32(0.0))
                th = jnp.maximum(
                    jnp.minimum(ky2[sl], by2) - jnp.maximum(ky1[sl], by1),
                    f32(0.0))
                inter = tw * th
                iou = inter / (kar[sl] - inter + bar)
                nk = jnp.where(iou <= f32(_OVERLAP), ksb[sl], _IMIN)
                ksb[sl] = nk
                acc = jnp.maximum(acc, nk)
            mm2 = jnp.max(acc)
            cv = jnp.where(lax.iota(i32, 16) == 0, count, _SH)
            iv = jnp.zeros((16,), i32) + ii
            plsc.store_scatter(keep_v, [cv], iv)
            return count + 1, mm2 > _IMIN, mm2

        acc0 = jnp.full((16,), _IMIN, i32)
        for k in range(_KV):
            acc0 = jnp.maximum(acc0, ksb[pl.ds(k * 16, 16)])
        mm0 = jnp.max(acc0)
        count, _, _ = lax.while_loop(cond, body, (i32(0), True, mm0))
        cntv[pl.ds(0, 16)] = jnp.where(lax.iota(i32, 16) == 0, count, 0)
        pltpu.sync_copy(cntv, cnt_h)
        pltpu.sync_copy(keep_v.at[pl.ds(0, _SH)],
                        keep_h.at[pl.ds(0, _SH)])


_SC_SCRATCH = [
        pltpu.VMEM((_SH,), jnp.int32),
        pltpu.VMEM((_SH,), jnp.float32),
        pltpu.VMEM((_SH,), jnp.float32),
        pltpu.VMEM((_SH,), jnp.float32),
        pltpu.VMEM((_SH,), jnp.float32),
        pltpu.VMEM((16,), jnp.int32),
        pltpu.VMEM((_PACK,), jnp.int32),
        pltpu.VMEM((16,), jnp.int32),
        pltpu.VMEM((_SH + 16,), jnp.int32),
        pltpu.VMEM_SHARED((_NW, _PACK), jnp.int32),
        pltpu.VMEM((_NW, _PACK), jnp.int32),
        pltpu.VMEM((_KPAD,), jnp.int32),
        pltpu.VMEM((_KPAD,), jnp.int32),
        pltpu.VMEM((_KPAD,), jnp.float32),
        pltpu.VMEM((_KPAD,), jnp.float32),
        pltpu.VMEM((_KPAD,), jnp.float32),
        pltpu.VMEM((_KPAD,), jnp.float32),
        pltpu.VMEM((_KPAD,), jnp.float32),
]


@functools.lru_cache(maxsize=1)
def _make_sc_call():
  return functools.partial(
    pl.kernel,
    out_type=(
        jax.ShapeDtypeStruct((_NPAD,), jnp.int32),
        jax.ShapeDtypeStruct((16,), jnp.int32),
    ),
    mesh=plsc.VectorSubcoreMesh(core_axis_name="c", subcore_axis_name="s",
                                num_cores=1),
    compiler_params=pltpu.CompilerParams(needs_layout_passes=False),
    scratch_types=_SC_SCRATCH,
)(_sc_kernel)


def kernel(loc, scores, dbox_list):
    f32 = jnp.float32
    locp = jnp.zeros((4, _NPAD), f32).at[:, :_N].set(loc.T).reshape(
        4, _ROWS, _LANES)
    dbxp = jnp.zeros((4, _NPAD), f32).at[:, :_N].set(dbox_list.T).reshape(
        4, _ROWS, _LANES)
    scop = jnp.full((_NPAD,), -jnp.inf, f32).at[:_N].set(scores).reshape(
        _ROWS, _LANES)
    boxes4, sbit, tj = pl.pallas_call(
        _tc_kernel,
        out_shape=(
            jax.ShapeDtypeStruct((4, _ROWS, _LANES), f32),
            jax.ShapeDtypeStruct((_ROWS, _LANES), jnp.int32),
            jax.ShapeDtypeStruct((1, 2), jnp.int32),
        ),
        out_specs=(
            pl.BlockSpec(),
            pl.BlockSpec(),
            pl.BlockSpec(memory_space=pltpu.SMEM),
        ),
    )(locp, scop, dbxp)
    b4 = boxes4.reshape(4, _NPAD)
    tj16 = jnp.zeros((16,), jnp.int32).at[0].set(tj[0, 0]).at[1].set(tj[0, 1])
    keep_p, cnt16 = _make_sc_call()(
        sbit.reshape(_NPAD), b4[0], b4[1], b4[2], b4[3], tj16)
    boxes = b4[:, :_N].T
    return boxes, keep_p[:_N], cnt16[0]
